# baseline (speedup 1.0000x reference)
import functools

import jax
import jax.numpy as jnp
from jax import lax
from jax.experimental import pallas as pl
from jax.experimental.pallas import tpu as pltpu
from jax.experimental.pallas import tpu_sc as plsc

_VOCAB = 1000000
_EMBED = 64
_BATCH = 16384
_NEG = 10

_NC = 2
_NS = 16
_NW = _NC * _NS
_CHUNK = _BATCH // _NW
_NB = 64
_NBLK = _CHUNK // _NB
_GROUPS = _NB // 16
_COLS = _NEG + 1


def _sc_dots(table, cidx, oidx, nidx):
    mesh = plsc.VectorSubcoreMesh(
        core_axis_name="c", subcore_axis_name="s",
        num_cores=_NC, num_subcores=_NS)

    @functools.partial(
        pl.kernel,
        out_type=jax.ShapeDtypeStruct((_NW, _COLS, _CHUNK), jnp.float32),
        mesh=mesh,
        compiler_params=pltpu.CompilerParams(
            needs_layout_passes=False, use_tc_tiling_on_sc=False),
        scratch_types=[
            pltpu.VMEM((_CHUNK,), jnp.int32),
            pltpu.VMEM((_CHUNK,), jnp.int32),
            pltpu.VMEM((_CHUNK * _NEG,), jnp.int32),
            pltpu.VMEM((2, _NB, _EMBED), jnp.float32),
            pltpu.VMEM((2, _NB, _EMBED), jnp.float32),
            pltpu.VMEM((2, _NB * _NEG, _EMBED), jnp.float32),
            pltpu.VMEM((_COLS, _CHUNK), jnp.float32),
            pltpu.SemaphoreType.DMA,
            pltpu.SemaphoreType.DMA,
        ],
    )
    def k(table_hbm, cidx_hbm, oidx_hbm, nidx_hbm, out_hbm,
          cidx_v, oidx_v, nidx_v, cbuf, obuf, nbuf, outv, sem0, sem1):
        sems = (sem0, sem1)
        wid = lax.axis_index("s") * _NC + lax.axis_index("c")
        base = wid * _CHUNK
        pltpu.sync_copy(cidx_hbm.at[pl.ds(base, _CHUNK)], cidx_v)
        pltpu.sync_copy(oidx_hbm.at[pl.ds(base, _CHUNK)], oidx_v)
        pltpu.sync_copy(nidx_hbm.at[pl.ds(base * _NEG, _CHUNK * _NEG)], nidx_v)

        def copies(blk, slot):
            off = blk * _NB
            cs = [
                pltpu.make_async_copy(
                    table_hbm.at[cidx_v.at[pl.ds(off, _NB)]],
                    cbuf.at[slot], sems[slot]),
                pltpu.make_async_copy(
                    table_hbm.at[oidx_v.at[pl.ds(off, _NB)]],
                    obuf.at[slot], sems[slot]),
            ]
            for j in range(_NB * _NEG // 128):
                cs.append(pltpu.make_async_copy(
                    table_hbm.at[nidx_v.at[pl.ds(off * _NEG + j * 128, 128)]],
                    nbuf.at[slot, pl.ds(j * 128, 128)], sems[slot]))
            return cs

        def fire(blk, slot):
            for c in copies(blk, slot):
                c.start()

        def drain(blk, slot):
            for c in copies(blk, slot):
                c.wait()

        lanes = lax.iota(jnp.int32, 16)
        lanes_neg = lanes * _NEG

        def compute(blk, slot):
            cb, ob, nb = cbuf.at[slot], obuf.at[slot], nbuf.at[slot]

            @pl.loop(0, _GROUPS)
            def _group(g):
                row_c = g * 16 + lanes
                rows_n = [lanes_neg + (g * (16 * _NEG) + kk)
                          for kk in range(_NEG)]
                acc = [jnp.zeros((16,), jnp.float32) for _ in range(_COLS)]
                for d in range(_EMBED):
                    ccol = (lanes + d) & (_EMBED - 1)
                    cd = plsc.load_gather(cb, [row_c, ccol])
                    od = plsc.load_gather(ob, [row_c, ccol])
                    acc[0] = acc[0] + cd * od
                    for kk in range(_NEG):
                        nv = plsc.load_gather(nb, [rows_n[kk], ccol])
                        acc[kk + 1] = acc[kk + 1] + cd * nv
                off = blk * _NB + g * 16
                for j in range(_COLS):
                    outv[j, pl.ds(off, 16)] = acc[j]

        fire(0, 0)

        @pl.loop(0, _NBLK // 2)
        def _pair(p):
            for slot in range(2):
                blk = p * 2 + slot
                nxt = blk + 1

                @pl.when(nxt < _NBLK)
                def _():
                    fire(nxt, 1 - slot)

                drain(blk, slot)
                compute(blk, slot)

        pltpu.sync_copy(outv, out_hbm.at[wid])

    return k(table, cidx, oidx, nidx)


_TW = 8192
_TSH = _TW.bit_length() - 1
_TGRID = (_VOCAB + 2 * _TW - 1) // (2 * _TW)
_VPAD = _TGRID * 2 * _TW


def _tc_transpose(table_t):
    def body(x0_ref, x1_ref, o_ref):
        eye = jnp.eye(_EMBED, dtype=jnp.float32)
        dn = (((0,), (0,)), ((), ()))
        o_ref[:, 0:_EMBED] = lax.dot_general(
            x0_ref[...], eye, dn, preferred_element_type=jnp.float32)
        o_ref[:, _EMBED:2 * _EMBED] = lax.dot_general(
            x1_ref[...], eye, dn, preferred_element_type=jnp.float32)

    out = pl.pallas_call(
        body,
        grid=(_TGRID,),
        in_specs=[
            pl.BlockSpec((_EMBED, _TW), lambda i: (0, 2 * i)),
            pl.BlockSpec(
                (_EMBED, _TW),
                lambda i: (0, jnp.minimum(2 * i + 1, _VOCAB // _TW))),
        ],
        out_specs=pl.BlockSpec((_TW, 2 * _EMBED), lambda i: (i, 0)),
        out_shape=jax.ShapeDtypeStruct((_VPAD // 2, 2 * _EMBED), jnp.float32),
    )(table_t, table_t)
    return out.reshape(_VPAD, _EMBED)


def _remap(v):
    j = v >> _TSH
    p = ((j >> 1) << _TSH) + (v & (_TW - 1))
    return (p << 1) + (j & 1)


def _tc_loss(dots):
    rows = _NW * _COLS

    def body(x_ref, o_ref):
        x = x_ref[...]
        r = lax.broadcasted_iota(jnp.int32, x.shape, 0)
        sign = jnp.where((r % _COLS) == 0, -1.0, 1.0)
        t = x * sign
        sp = jnp.maximum(t, 0.0) + jnp.log1p(jnp.exp(-jnp.abs(t)))
        o_ref[0, 0] = jnp.sum(sp) * (1.0 / _BATCH)

    return pl.pallas_call(
        body,
        out_shape=jax.ShapeDtypeStruct((1, 1), jnp.float32),
        out_specs=pl.BlockSpec(memory_space=pltpu.SMEM),
    )(dots.reshape(rows, _CHUNK))


def kernel(center_words, outside_words, word_embeddings):
    cidx = _remap(center_words.astype(jnp.int32))
    oidx = _remap(outside_words.astype(jnp.int32))
    neg = jax.random.randint(jax.random.key(42), (_BATCH, _NEG), 0, _VOCAB)
    nidx = _remap(neg.reshape(-1).astype(jnp.int32))
    table_lin = _tc_transpose(word_embeddings.T)
    dots = _sc_dots(table_lin, cidx, oidx, nidx)
    loss = _tc_loss(dots)
    return loss[0, 0]


# speedup vs baseline: 1.1035x; 1.1035x over previous
import functools

import jax
import jax.numpy as jnp
from jax import lax
from jax.experimental import pallas as pl
from jax.experimental.pallas import tpu as pltpu
from jax.experimental.pallas import tpu_sc as plsc

_VOCAB = 1000000
_EMBED = 64
_BATCH = 16384
_NEG = 10

_NC = 2
_NS = 16
_NW = _NC * _NS
_CHUNK = _BATCH // _NW
_NB = 64
_NBLK = _CHUNK // _NB
_GROUPS = _NB // 16
_COLS = _NEG + 1


def _sc_dots(table, cidx, oidx, nidx):
    mesh = plsc.VectorSubcoreMesh(
        core_axis_name="c", subcore_axis_name="s",
        num_cores=_NC, num_subcores=_NS)

    @functools.partial(
        pl.kernel,
        out_type=jax.ShapeDtypeStruct((_NW, _COLS, _CHUNK), jnp.float32),
        mesh=mesh,
        compiler_params=pltpu.CompilerParams(
            needs_layout_passes=False, use_tc_tiling_on_sc=False),
        scratch_types=[
            pltpu.VMEM((_CHUNK,), jnp.int32),
            pltpu.VMEM((_CHUNK,), jnp.int32),
            pltpu.VMEM((_CHUNK * _NEG,), jnp.int32),
            pltpu.VMEM((2, _NB, _EMBED), jnp.float32),
            pltpu.VMEM((2, _NB, _EMBED), jnp.float32),
            pltpu.VMEM((2, _NB * _NEG, _EMBED), jnp.float32),
            pltpu.VMEM((_COLS, _CHUNK), jnp.float32),
            pltpu.SemaphoreType.DMA,
            pltpu.SemaphoreType.DMA,
        ],
    )
    def k(table_hbm, cidx_hbm, oidx_hbm, nidx_hbm, out_hbm,
          cidx_v, oidx_v, nidx_v, cbuf, obuf, nbuf, outv, sem0, sem1):
        sems = (sem0, sem1)
        wid = lax.axis_index("s") * _NC + lax.axis_index("c")
        base = wid * _CHUNK
        pltpu.sync_copy(cidx_hbm.at[pl.ds(base, _CHUNK)], cidx_v)
        pltpu.sync_copy(oidx_hbm.at[pl.ds(base, _CHUNK)], oidx_v)
        pltpu.sync_copy(nidx_hbm.at[pl.ds(base * _NEG, _CHUNK * _NEG)], nidx_v)

        def copies(blk, slot):
            off = blk * _NB
            cs = [
                pltpu.make_async_copy(
                    table_hbm.at[cidx_v.at[pl.ds(off, _NB)]],
                    cbuf.at[slot], sems[slot]),
                pltpu.make_async_copy(
                    table_hbm.at[oidx_v.at[pl.ds(off, _NB)]],
                    obuf.at[slot], sems[slot]),
            ]
            for j in range(_NB * _NEG // 128):
                cs.append(pltpu.make_async_copy(
                    table_hbm.at[nidx_v.at[pl.ds(off * _NEG + j * 128, 128)]],
                    nbuf.at[slot, pl.ds(j * 128, 128)], sems[slot]))
            return cs

        def fire(blk, slot):
            for c in copies(blk, slot):
                c.start()

        def drain(blk, slot):
            for c in copies(blk, slot):
                c.wait()

        lanes = lax.iota(jnp.int32, 16)
        lanes_neg = lanes * _NEG

        def compute(blk, slot):
            cb, ob, nb = cbuf.at[slot], obuf.at[slot], nbuf.at[slot]

            @pl.loop(0, _GROUPS)
            def _group(g):
                row_c = g * 16 + lanes
                rows_n = [lanes_neg + (g * (16 * _NEG) + kk)
                          for kk in range(_NEG)]
                acc = [jnp.zeros((16,), jnp.float32) for _ in range(_COLS)]
                for d in range(_EMBED):
                    ccol = (lanes + d) & (_EMBED - 1)
                    cd = plsc.load_gather(cb, [row_c, ccol])
                    od = plsc.load_gather(ob, [row_c, ccol])
                    acc[0] = acc[0] + cd * od
                    for kk in range(_NEG):
                        nv = plsc.load_gather(nb, [rows_n[kk], ccol])
                        acc[kk + 1] = acc[kk + 1] + cd * nv
                off = blk * _NB + g * 16
                for j in range(_COLS):
                    outv[j, pl.ds(off, 16)] = acc[j]

        fire(0, 0)

        @pl.loop(0, _NBLK // 2)
        def _pair(p):
            for slot in range(2):
                blk = p * 2 + slot
                nxt = blk + 1

                @pl.when(nxt < _NBLK)
                def _():
                    fire(nxt, 1 - slot)

                drain(blk, slot)
                compute(blk, slot)

        pltpu.sync_copy(outv, out_hbm.at[wid])

    return k(table, cidx, oidx, nidx)


_TW = 16384
_TSH = _TW.bit_length() - 1
_TGRID = (_VOCAB + 2 * _TW - 1) // (2 * _TW)
_VPAD = _TGRID * 2 * _TW


def _tc_transpose(table_t):
    def body(x0_ref, x1_ref, o_ref):
        eye = jnp.eye(_EMBED, dtype=jnp.float32)
        dn = (((0,), (0,)), ((), ()))
        o_ref[:, 0:_EMBED] = lax.dot_general(
            x0_ref[...], eye, dn, preferred_element_type=jnp.float32)
        o_ref[:, _EMBED:2 * _EMBED] = lax.dot_general(
            x1_ref[...], eye, dn, preferred_element_type=jnp.float32)

    out = pl.pallas_call(
        body,
        grid=(_TGRID,),
        in_specs=[
            pl.BlockSpec((_EMBED, _TW), lambda i: (0, 2 * i)),
            pl.BlockSpec(
                (_EMBED, _TW),
                lambda i: (0, jnp.minimum(2 * i + 1, _VOCAB // _TW))),
        ],
        out_specs=pl.BlockSpec((_TW, 2 * _EMBED), lambda i: (i, 0)),
        out_shape=jax.ShapeDtypeStruct((_VPAD // 2, 2 * _EMBED), jnp.float32),
    )(table_t, table_t)
    return out.reshape(_VPAD, _EMBED)


def _remap(v):
    j = v >> _TSH
    p = ((j >> 1) << _TSH) + (v & (_TW - 1))
    return (p << 1) + (j & 1)


def _tc_loss(dots):
    rows = _NW * _COLS

    def body(x_ref, o_ref):
        x = x_ref[...]
        r = lax.broadcasted_iota(jnp.int32, x.shape, 0)
        sign = jnp.where((r % _COLS) == 0, -1.0, 1.0)
        t = x * sign
        sp = jnp.maximum(t, 0.0) + jnp.log1p(jnp.exp(-jnp.abs(t)))
        o_ref[0, 0] = jnp.sum(sp) * (1.0 / _BATCH)

    return pl.pallas_call(
        body,
        out_shape=jax.ShapeDtypeStruct((1, 1), jnp.float32),
        out_specs=pl.BlockSpec(memory_space=pltpu.SMEM),
    )(dots.reshape(rows, _CHUNK))


def kernel(center_words, outside_words, word_embeddings):
    cidx = _remap(center_words.astype(jnp.int32))
    oidx = _remap(outside_words.astype(jnp.int32))
    neg = jax.random.randint(
        jax.random.key(42), (_BATCH * _NEG,), 0, _VOCAB)
    nidx = _remap(neg.astype(jnp.int32))
    table_lin = _tc_transpose(word_embeddings.T)
    dots = _sc_dots(table_lin, cidx, oidx, nidx)
    loss = _tc_loss(dots)
    return loss[0, 0]
